# baseline (speedup 1.0000x reference)
import functools

import jax
import jax.numpy as jnp
from jax import lax
from jax.experimental import pallas as pl
from jax.experimental.pallas import tpu as pltpu
from jax.experimental.pallas import tpu_sc as plsc

VOCAB = 100000
HIDDEN = 2048
N_TOKENS = 4 * 4096

NUM_CORES = 2
NUM_SUBCORES = 16
NW = NUM_CORES * NUM_SUBCORES
PER_W = N_TOKENS // NW
CHUNK = 8
NCHUNK = PER_W // CHUNK
NBUF = 4
NROUNDS = NCHUNK // NBUF
LOOKAHEAD = 3

_mesh = plsc.VectorSubcoreMesh(core_axis_name="c", subcore_axis_name="s")


@functools.partial(
    pl.kernel,
    out_type=jax.ShapeDtypeStruct((N_TOKENS, HIDDEN), jnp.float32),
    mesh=_mesh,
    scratch_types=[
        pltpu.VMEM((PER_W,), jnp.int32),
        [pltpu.VMEM((CHUNK, HIDDEN), jnp.float32) for _ in range(NBUF)],
        [pltpu.SemaphoreType.DMA for _ in range(NBUF)],
        [pltpu.SemaphoreType.DMA for _ in range(NBUF)],
    ],
)
def _gather_kernel(ids_hbm, table_hbm, out_hbm, idx_v, bufs, gsems, ssems):
    wid = lax.axis_index("s") * NUM_CORES + lax.axis_index("c")
    base = wid * PER_W
    pltpu.sync_copy(ids_hbm.at[pl.ds(base, PER_W)], idx_v)

    def start_gather(g, b):
        pltpu.async_copy(
            table_hbm.at[idx_v.at[pl.ds(g * CHUNK, CHUNK)]], bufs[b], gsems[b]
        )

    def wait_gather(b):
        pltpu.make_async_copy(
            table_hbm.at[idx_v.at[pl.ds(0, CHUNK)]], bufs[b], gsems[b]
        ).wait()

    def start_scatter(g, b):
        pltpu.async_copy(bufs[b], out_hbm.at[pl.ds(base + g * CHUNK, CHUNK)],
                         ssems[b])

    def wait_scatter(b):
        pltpu.make_async_copy(
            bufs[b], out_hbm.at[pl.ds(base, CHUNK)], ssems[b]
        ).wait()

    for g in range(LOOKAHEAD):
        start_gather(g, g % NBUF)

    def round_body(i, carry):
        g0 = i * NBUF
        for b in range(NBUF):
            g = g0 + b
            ga = g + LOOKAHEAD
            ba = (b + LOOKAHEAD) % NBUF

            @pl.when(jnp.logical_and(ga >= NBUF, ga < NCHUNK))
            def _():
                wait_scatter(ba)

            @pl.when(ga < NCHUNK)
            def _():
                start_gather(ga, ba)

            wait_gather(b)
            start_scatter(g, b)
        return carry

    lax.fori_loop(0, NROUNDS, round_body, 0)

    for b in range(NBUF):
        wait_scatter(b)


def kernel(input_ids, lookup_table):
    flat_ids = input_ids.reshape(N_TOKENS).astype(jnp.int32)
    out = _gather_kernel(flat_ids, lookup_table)
    return out.reshape(input_ids.shape + (HIDDEN,))


# speedup vs baseline: 1.0233x; 1.0233x over previous
import functools

import jax
import jax.numpy as jnp
from jax import lax
from jax.experimental import pallas as pl
from jax.experimental.pallas import tpu as pltpu
from jax.experimental.pallas import tpu_sc as plsc

VOCAB = 100000
HIDDEN = 2048
N_TOKENS = 4 * 4096

NUM_CORES = 2
NUM_SUBCORES = 16
NW = NUM_CORES * NUM_SUBCORES
PER_W = N_TOKENS // NW
CHUNK = 8
NCHUNK = PER_W // CHUNK
NBUF = 4
NROUNDS = NCHUNK // NBUF
LOOKAHEAD = 3

_mesh = plsc.VectorSubcoreMesh(core_axis_name="c", subcore_axis_name="s")


@functools.partial(
    pl.kernel,
    out_type=jax.ShapeDtypeStruct((N_TOKENS, HIDDEN), jnp.float32),
    mesh=_mesh,
    scratch_types=[
        pltpu.VMEM((PER_W,), jnp.int32),
        [pltpu.VMEM((CHUNK, HIDDEN), jnp.float32) for _ in range(NBUF)],
        pltpu.VMEM_SHARED((NUM_SUBCORES, 2, CHUNK, HIDDEN), jnp.float32),
        [pltpu.SemaphoreType.DMA for _ in range(NBUF)],
        [pltpu.SemaphoreType.DMA for _ in range(NBUF)],
        [pltpu.SemaphoreType.DMA for _ in range(2)],
    ],
)
def _gather_kernel(ids_hbm, table_hbm, out_hbm, idx_v, bufs, sp, gsems,
                   h1sems, ssems):
    cid = lax.axis_index("c")
    sid = lax.axis_index("s")
    wid = sid * NUM_CORES + cid
    base = wid * PER_W
    pltpu.sync_copy(ids_hbm.at[pl.ds(base, PER_W)], idx_v)

    def start_gather(g, b):
        pltpu.async_copy(
            table_hbm.at[idx_v.at[pl.ds(g * CHUNK, CHUNK)]], bufs[b], gsems[b]
        )

    def wait_gather(b):
        pltpu.make_async_copy(
            table_hbm.at[idx_v.at[pl.ds(0, CHUNK)]], bufs[b], gsems[b]
        ).wait()

    def start_hop1(b):
        pltpu.async_copy(bufs[b], sp.at[sid, b % 2], h1sems[b])

    def wait_hop1(b):
        pltpu.make_async_copy(bufs[b], sp.at[sid, b % 2], h1sems[b]).wait()

    def start_hop2(g, b):
        pltpu.async_copy(sp.at[sid, b % 2],
                         out_hbm.at[pl.ds(base + g * CHUNK, CHUNK)],
                         ssems[b % 2])

    def wait_hop2(b):
        pltpu.make_async_copy(
            sp.at[sid, b % 2], out_hbm.at[pl.ds(base, CHUNK)], ssems[b % 2]
        ).wait()

    for g in range(LOOKAHEAD):
        start_gather(g, g % NBUF)

    def round_body(i, carry):
        g0 = i * NBUF
        for b in range(NBUF):
            g = g0 + b
            bp = (b - 1) % NBUF
            ga = g + LOOKAHEAD
            ba = (b + LOOKAHEAD) % NBUF

            @pl.when(g >= 1)
            def _():
                wait_hop1(bp)
                start_hop2(g - 1, bp)

            @pl.when(ga < NCHUNK)
            def _():
                start_gather(ga, ba)

            @pl.when(g >= 2)
            def _():
                wait_hop2(b)

            wait_gather(b)
            start_hop1(b)
        return carry

    lax.fori_loop(0, NROUNDS, round_body, 0)

    bl = (NCHUNK - 1) % NBUF
    wait_hop1(bl)
    start_hop2(NCHUNK - 1, bl)
    wait_hop2(NCHUNK - 2)
    wait_hop2(NCHUNK - 1)


def kernel(input_ids, lookup_table):
    flat_ids = input_ids.reshape(N_TOKENS).astype(jnp.int32)
    out = _gather_kernel(flat_ids, lookup_table)
    return out.reshape(input_ids.shape + (HIDDEN,))
